# baseline (speedup 1.0000x reference)
import functools

import jax
import jax.numpy as jnp
from jax import lax
from jax.experimental import pallas as pl
from jax.experimental.pallas import tpu as pltpu
from jax.experimental.pallas import tpu_sc as plsc

N_NODES = 10000
N_EDGES = 160000
D_HALF = 128

NC = 2
NS = 16
CHUNK = 112
E_PER_SUBCORE = 10080
NCHUNKS = E_PER_SUBCORE // CHUNK
E_PAD = E_PER_SUBCORE - N_EDGES // NS
ACC_ROWS = 10016
WB_ROWS = 80
WB_CHUNKS = N_NODES // WB_ROWS
WB_ITERS = (WB_CHUNKS + NS - 1) // NS


def _sc_gather_scatter(feat2, pk):
    mesh = plsc.VectorSubcoreMesh(
        core_axis_name="c", subcore_axis_name="s",
        num_cores=NC, num_subcores=NS)

    @functools.partial(
        pl.kernel,
        out_type=jax.ShapeDtypeStruct((NC * N_NODES, D_HALF), jnp.float32),
        mesh=mesh,
        scratch_types=[
            pltpu.VMEM_SHARED((ACC_ROWS, D_HALF), jnp.float32),
            pltpu.VMEM((NCHUNKS, CHUNK), jnp.int32),
            pltpu.VMEM((CHUNK,), jnp.int32),
            pltpu.VMEM((CHUNK,), jnp.int32),
            pltpu.VMEM((CHUNK,), jnp.int32),
            pltpu.VMEM((CHUNK,), jnp.int32),
            pltpu.VMEM((CHUNK, D_HALF), jnp.float32),
            pltpu.VMEM((CHUNK, D_HALF), jnp.float32),
            pltpu.SemaphoreType.DMA,
            pltpu.SemaphoreType.DMA,
            pltpu.SemaphoreType.DMA,
            pltpu.SemaphoreType.DMA,
        ],
    )
    def k(feat_hbm, pk_hbm, out_hbm,
          acc, pk_v, sidx0, sidx1, didx0, didx1, rows0, rows1,
          gsem0, gsem1, ssem0, ssem1):
        c = lax.axis_index("c")
        s = lax.axis_index("s")

        def unpack(i, sidx_b, didx_b):
            for q in range(CHUNK // 16):
                v = pk_v[i, pl.ds(q * 16, 16)]
                sidx_b[pl.ds(q * 16, 16)] = ((v & 0xFFFF) << 1) | c
                didx_b[pl.ds(q * 16, 16)] = v >> 16

        zv = jnp.zeros((16,), jnp.float32)

        def zrow(i, carry):
            for jj in range(D_HALF // 16):
                rows0[i, pl.ds(jj * 16, 16)] = zv
            return carry

        lax.fori_loop(0, WB_ROWS, zrow, 0)
        for i in range(WB_ITERS):
            idx = s + i * NS

            @pl.when(idx < WB_CHUNKS)
            def _():
                pltpu.sync_copy(rows0.at[pl.ds(0, WB_ROWS)],
                                acc.at[pl.ds(idx * WB_ROWS, WB_ROWS)])

        pltpu.sync_copy(pk_hbm.at[s], pk_v)

        plsc.subcore_barrier()

        def do_edges(feat_hbm):
            unpack(0, sidx0, didx0)
            unpack(1, sidx1, didx1)
            pltpu.async_copy(feat_hbm.at[sidx0], rows0, gsem0)
            pltpu.async_copy(feat_hbm.at[sidx1], rows1, gsem1)

            def step(i, rows_b, gsem_b, sidx_b, didx_b):
                pltpu.make_async_copy(
                    feat_hbm.at[sidx_b], rows_b, gsem_b).wait()

                pltpu.sync_copy(rows_b, acc.at[didx_b], add=True)

                @pl.when(i + 2 < NCHUNKS)
                def _():
                    unpack(i + 2, sidx_b, didx_b)
                    pltpu.async_copy(feat_hbm.at[sidx_b], rows_b, gsem_b)

            def outer(t, carry):
                step(2 * t, rows0, gsem0, sidx0, didx0)
                step(2 * t + 1, rows1, gsem1, sidx1, didx1)
                return carry

            lax.fori_loop(0, NCHUNKS // 2, outer, 0)

        do_edges(feat_hbm)

        plsc.subcore_barrier()

        for i in range(WB_ITERS):
            idx = s + i * NS

            @pl.when(idx < WB_CHUNKS)
            def _():
                off = idx * WB_ROWS
                pltpu.sync_copy(acc.at[pl.ds(off, WB_ROWS)],
                                rows0.at[pl.ds(0, WB_ROWS)])
                pltpu.sync_copy(rows0.at[pl.ds(0, WB_ROWS)],
                                out_hbm.at[pl.ds(c * N_NODES + off, WB_ROWS)])

    return k(feat2, pk)


def _tc_body(x_ref, wt_ref, b_ref, o_ref):
    acc = jnp.dot(x_ref[0], wt_ref[:D_HALF, :],
                  preferred_element_type=jnp.float32)
    acc += jnp.dot(x_ref[1], wt_ref[D_HALF:, :],
                   preferred_element_type=jnp.float32)
    o_ref[...] = jnp.maximum(acc + b_ref[...], 0.0)


def _tc_linear_relu(agg2, wt, b2):
    blk = 2000
    grid = N_NODES // blk
    return pl.pallas_call(
        _tc_body,
        grid=(grid,),
        in_specs=[
            pl.BlockSpec((2, blk, D_HALF), lambda i: (0, i, 0)),
            pl.BlockSpec((2 * D_HALF, 2 * D_HALF), lambda i: (0, 0)),
            pl.BlockSpec((1, 2 * D_HALF), lambda i: (0, 0)),
        ],
        out_specs=pl.BlockSpec((blk, 2 * D_HALF), lambda i: (i, 0)),
        out_shape=jax.ShapeDtypeStruct((N_NODES, 2 * D_HALF), jnp.float32),
    )(agg2, wt, b2)


def kernel(feature, edge_index, W, b):
    src = edge_index[0].astype(jnp.int32)
    dst = edge_index[1].astype(jnp.int32)
    pk = (src | (dst << 16)).reshape(NS, N_EDGES // NS)
    pk = jnp.concatenate(
        [pk, jnp.full((NS, E_PAD), N_NODES << 16, jnp.int32)], axis=1)
    pk = pk.reshape(NS, NCHUNKS, CHUNK)
    feat2 = feature.reshape(2 * N_NODES, D_HALF)
    agg2 = _sc_gather_scatter(feat2, pk)
    return _tc_linear_relu(agg2.reshape(NC, N_NODES, D_HALF), W.T,
                           b.reshape(1, 2 * D_HALF))


# speedup vs baseline: 1.0421x; 1.0421x over previous
import functools

import jax
import jax.numpy as jnp
from jax import lax
from jax.experimental import pallas as pl
from jax.experimental.pallas import tpu as pltpu
from jax.experimental.pallas import tpu_sc as plsc

N_NODES = 10000
N_EDGES = 160000
D_HALF = 128

NC = 2
NS = 16
CHUNK = 112
E_PER_SUBCORE = 10080
NCHUNKS = E_PER_SUBCORE // CHUNK
E_PAD = E_PER_SUBCORE - N_EDGES // NS
ACC_ROWS = 10016
WB_ROWS = 80
WB_CHUNKS = N_NODES // WB_ROWS
WB_ITERS = (WB_CHUNKS + NS - 1) // NS


def _sc_gather_scatter(flo, fhi, pk):
    mesh = plsc.VectorSubcoreMesh(
        core_axis_name="c", subcore_axis_name="s",
        num_cores=NC, num_subcores=NS)

    @functools.partial(
        pl.kernel,
        out_type=jax.ShapeDtypeStruct((NC * N_NODES, D_HALF), jnp.float32),
        mesh=mesh,
        scratch_types=[
            pltpu.VMEM_SHARED((ACC_ROWS, D_HALF), jnp.float32),
            pltpu.VMEM((NCHUNKS, CHUNK), jnp.int32),
            pltpu.VMEM((CHUNK,), jnp.int32),
            pltpu.VMEM((CHUNK,), jnp.int32),
            pltpu.VMEM((CHUNK,), jnp.int32),
            pltpu.VMEM((CHUNK,), jnp.int32),
            pltpu.VMEM((CHUNK, D_HALF), jnp.float32),
            pltpu.VMEM((CHUNK, D_HALF), jnp.float32),
            pltpu.SemaphoreType.DMA,
            pltpu.SemaphoreType.DMA,
            pltpu.SemaphoreType.DMA,
            pltpu.SemaphoreType.DMA,
        ],
    )
    def k(flo_hbm, fhi_hbm, pk_hbm, out_hbm,
          acc, pk_v, sidx0, sidx1, didx0, didx1, rows0, rows1,
          gsem0, gsem1, ssem0, ssem1):
        c = lax.axis_index("c")
        s = lax.axis_index("s")

        def unpack(i, sidx_b, didx_b):
            for q in range(CHUNK // 16):
                v = pk_v[i, pl.ds(q * 16, 16)]
                sidx_b[pl.ds(q * 16, 16)] = v & 0xFFFF
                didx_b[pl.ds(q * 16, 16)] = v >> 16

        zv = jnp.zeros((16,), jnp.float32)

        def zrow(i, carry):
            for jj in range(D_HALF // 16):
                rows0[i, pl.ds(jj * 16, 16)] = zv
            return carry

        lax.fori_loop(0, WB_ROWS, zrow, 0)
        for i in range(WB_ITERS):
            idx = s + i * NS

            @pl.when(idx < WB_CHUNKS)
            def _():
                pltpu.async_copy(rows0.at[pl.ds(0, WB_ROWS)],
                                 acc.at[pl.ds(idx * WB_ROWS, WB_ROWS)], ssem0)

        pltpu.sync_copy(pk_hbm.at[s], pk_v)
        unpack(0, sidx0, didx0)
        unpack(1, sidx1, didx1)

        for i in range(WB_ITERS):
            idx = s + i * NS

            @pl.when(idx < WB_CHUNKS)
            def _():
                pltpu.make_async_copy(
                    rows0.at[pl.ds(0, WB_ROWS)],
                    acc.at[pl.ds(idx * WB_ROWS, WB_ROWS)], ssem0).wait()

        def do_edges(feat_hbm):
            pltpu.async_copy(feat_hbm.at[sidx0], rows0, gsem0)
            pltpu.async_copy(feat_hbm.at[sidx1], rows1, gsem1)

            plsc.subcore_barrier()

            def step(i, rows_b, gsem_b, sidx_b, didx_b):
                pltpu.make_async_copy(
                    feat_hbm.at[sidx_b], rows_b, gsem_b).wait()

                pltpu.sync_copy(rows_b, acc.at[didx_b], add=True)

                @pl.when(i + 2 < NCHUNKS)
                def _():
                    unpack(i + 2, sidx_b, didx_b)
                    pltpu.async_copy(feat_hbm.at[sidx_b], rows_b, gsem_b)

            def outer(t, carry):
                step(2 * t, rows0, gsem0, sidx0, didx0)
                step(2 * t + 1, rows1, gsem1, sidx1, didx1)
                return carry

            lax.fori_loop(0, NCHUNKS // 2, outer, 0)

        @pl.when(c == 0)
        def _():
            do_edges(flo_hbm)

        @pl.when(c == 1)
        def _():
            do_edges(fhi_hbm)

        plsc.subcore_barrier()

        bufs = (rows0, rows1)
        sems = (ssem0, ssem1)
        for i in range(WB_ITERS):
            idx = s + i * NS
            rb, sb = bufs[i % 2], sems[i % 2]

            if i >= 2:
                idx_prev = s + (i - 2) * NS

                @pl.when(idx_prev < WB_CHUNKS)
                def _():
                    off_p = idx_prev * WB_ROWS
                    pltpu.make_async_copy(
                        rb.at[pl.ds(0, WB_ROWS)],
                        out_hbm.at[pl.ds(c * N_NODES + off_p, WB_ROWS)],
                        sb).wait()

            @pl.when(idx < WB_CHUNKS)
            def _():
                off = idx * WB_ROWS
                pltpu.sync_copy(acc.at[pl.ds(off, WB_ROWS)],
                                rb.at[pl.ds(0, WB_ROWS)])
                pltpu.async_copy(rb.at[pl.ds(0, WB_ROWS)],
                                 out_hbm.at[pl.ds(c * N_NODES + off, WB_ROWS)],
                                 sb)

        for i in range(WB_ITERS - 2, WB_ITERS):
            idx = s + i * NS
            rb, sb = bufs[i % 2], sems[i % 2]

            @pl.when(idx < WB_CHUNKS)
            def _():
                off = idx * WB_ROWS
                pltpu.make_async_copy(
                    rb.at[pl.ds(0, WB_ROWS)],
                    out_hbm.at[pl.ds(c * N_NODES + off, WB_ROWS)], sb).wait()

    return k(flo, fhi, pk)


def _tc_body(x_ref, wt_ref, b_ref, o_ref):
    acc = jnp.dot(x_ref[0], wt_ref[:D_HALF, :],
                  preferred_element_type=jnp.float32)
    acc += jnp.dot(x_ref[1], wt_ref[D_HALF:, :],
                   preferred_element_type=jnp.float32)
    o_ref[...] = jnp.maximum(acc + b_ref[...], 0.0)


def _tc_linear_relu(agg2, wt, b2):
    blk = 2000
    grid = N_NODES // blk
    return pl.pallas_call(
        _tc_body,
        grid=(grid,),
        in_specs=[
            pl.BlockSpec((2, blk, D_HALF), lambda i: (0, i, 0)),
            pl.BlockSpec((2 * D_HALF, 2 * D_HALF), lambda i: (0, 0)),
            pl.BlockSpec((1, 2 * D_HALF), lambda i: (0, 0)),
        ],
        out_specs=pl.BlockSpec((blk, 2 * D_HALF), lambda i: (i, 0)),
        out_shape=jax.ShapeDtypeStruct((N_NODES, 2 * D_HALF), jnp.float32),
    )(agg2, wt, b2)


def kernel(feature, edge_index, W, b):
    src = edge_index[0].astype(jnp.int32)
    dst = edge_index[1].astype(jnp.int32)
    pk = (src | (dst << 16)).reshape(NS, N_EDGES // NS)
    pk = jnp.concatenate(
        [pk, jnp.full((NS, E_PAD), N_NODES << 16, jnp.int32)], axis=1)
    pk = pk.reshape(NS, NCHUNKS, CHUNK)
    flo = feature[:, :D_HALF]
    fhi = feature[:, D_HALF:]
    agg2 = _sc_gather_scatter(flo, fhi, pk)
    return _tc_linear_relu(agg2.reshape(NC, N_NODES, D_HALF), W.T,
                           b.reshape(1, 2 * D_HALF))


# speedup vs baseline: 1.0516x; 1.0091x over previous
import functools

import jax
import jax.numpy as jnp
from jax import lax
from jax.experimental import pallas as pl
from jax.experimental.pallas import tpu as pltpu
from jax.experimental.pallas import tpu_sc as plsc

N_NODES = 10000
N_EDGES = 160000
D_HALF = 128

NC = 2
NS = 16
CHUNK = 120
E_PER_SUBCORE = 10080
NCHUNKS = E_PER_SUBCORE // CHUNK
_UNPACK_OFFS = list(range(0, CHUNK - 15, 16))
if CHUNK % 16:
    _UNPACK_OFFS.append(CHUNK - 16)
E_PAD = E_PER_SUBCORE - N_EDGES // NS
ACC_ROWS = 10016
WB_ROWS = 80
WB_CHUNKS = N_NODES // WB_ROWS
WB_ITERS = (WB_CHUNKS + NS - 1) // NS


def _sc_gather_scatter(flo, fhi, pk):
    mesh = plsc.VectorSubcoreMesh(
        core_axis_name="c", subcore_axis_name="s",
        num_cores=NC, num_subcores=NS)

    @functools.partial(
        pl.kernel,
        out_type=jax.ShapeDtypeStruct((NC * N_NODES, D_HALF), jnp.float32),
        mesh=mesh,
        scratch_types=[
            pltpu.VMEM_SHARED((ACC_ROWS, D_HALF), jnp.float32),
            pltpu.VMEM((NCHUNKS, CHUNK), jnp.int32),
            pltpu.VMEM((CHUNK,), jnp.int32),
            pltpu.VMEM((CHUNK,), jnp.int32),
            pltpu.VMEM((CHUNK,), jnp.int32),
            pltpu.VMEM((CHUNK,), jnp.int32),
            pltpu.VMEM((CHUNK, D_HALF), jnp.float32),
            pltpu.VMEM((CHUNK, D_HALF), jnp.float32),
            pltpu.SemaphoreType.DMA,
            pltpu.SemaphoreType.DMA,
            pltpu.SemaphoreType.DMA,
            pltpu.SemaphoreType.DMA,
        ],
    )
    def k(flo_hbm, fhi_hbm, pk_hbm, out_hbm,
          acc, pk_v, sidx0, sidx1, didx0, didx1, rows0, rows1,
          gsem0, gsem1, ssem0, ssem1):
        c = lax.axis_index("c")
        s = lax.axis_index("s")

        def unpack(i, sidx_b, didx_b):
            for off in _UNPACK_OFFS:
                v = pk_v[i, pl.ds(off, 16)]
                sidx_b[pl.ds(off, 16)] = v & 0xFFFF
                didx_b[pl.ds(off, 16)] = v >> 16

        zv = jnp.zeros((16,), jnp.float32)

        def zrow(i, carry):
            for jj in range(D_HALF // 16):
                rows0[i, pl.ds(jj * 16, 16)] = zv
            return carry

        lax.fori_loop(0, WB_ROWS, zrow, 0)
        for i in range(WB_ITERS):
            idx = s + i * NS

            @pl.when(idx < WB_CHUNKS)
            def _():
                pltpu.async_copy(rows0.at[pl.ds(0, WB_ROWS)],
                                 acc.at[pl.ds(idx * WB_ROWS, WB_ROWS)], ssem0)

        pltpu.sync_copy(pk_hbm.at[s], pk_v)
        unpack(0, sidx0, didx0)
        unpack(1, sidx1, didx1)

        for i in range(WB_ITERS):
            idx = s + i * NS

            @pl.when(idx < WB_CHUNKS)
            def _():
                pltpu.make_async_copy(
                    rows0.at[pl.ds(0, WB_ROWS)],
                    acc.at[pl.ds(idx * WB_ROWS, WB_ROWS)], ssem0).wait()

        def do_edges(feat_hbm):
            pltpu.async_copy(feat_hbm.at[sidx0], rows0, gsem0)
            pltpu.async_copy(feat_hbm.at[sidx1], rows1, gsem1)

            plsc.subcore_barrier()

            def step(i, rows_b, gsem_b, sidx_b, didx_b):
                pltpu.make_async_copy(
                    feat_hbm.at[sidx_b], rows_b, gsem_b).wait()

                pltpu.sync_copy(rows_b, acc.at[didx_b], add=True)

                @pl.when(i + 2 < NCHUNKS)
                def _():
                    unpack(i + 2, sidx_b, didx_b)
                    pltpu.async_copy(feat_hbm.at[sidx_b], rows_b, gsem_b)

            def outer(t, carry):
                step(2 * t, rows0, gsem0, sidx0, didx0)
                step(2 * t + 1, rows1, gsem1, sidx1, didx1)
                return carry

            lax.fori_loop(0, NCHUNKS // 2, outer, 0)

        @pl.when(c == 0)
        def _():
            do_edges(flo_hbm)

        @pl.when(c == 1)
        def _():
            do_edges(fhi_hbm)

        plsc.subcore_barrier()

        bufs = (rows0, rows1)
        sems = (ssem0, ssem1)
        for i in range(WB_ITERS):
            idx = s + i * NS
            rb, sb = bufs[i % 2], sems[i % 2]

            if i >= 2:
                idx_prev = s + (i - 2) * NS

                @pl.when(idx_prev < WB_CHUNKS)
                def _():
                    off_p = idx_prev * WB_ROWS
                    pltpu.make_async_copy(
                        rb.at[pl.ds(0, WB_ROWS)],
                        out_hbm.at[pl.ds(c * N_NODES + off_p, WB_ROWS)],
                        sb).wait()

            @pl.when(idx < WB_CHUNKS)
            def _():
                off = idx * WB_ROWS
                pltpu.sync_copy(acc.at[pl.ds(off, WB_ROWS)],
                                rb.at[pl.ds(0, WB_ROWS)])
                pltpu.async_copy(rb.at[pl.ds(0, WB_ROWS)],
                                 out_hbm.at[pl.ds(c * N_NODES + off, WB_ROWS)],
                                 sb)

        for i in range(WB_ITERS - 2, WB_ITERS):
            idx = s + i * NS
            rb, sb = bufs[i % 2], sems[i % 2]

            @pl.when(idx < WB_CHUNKS)
            def _():
                off = idx * WB_ROWS
                pltpu.make_async_copy(
                    rb.at[pl.ds(0, WB_ROWS)],
                    out_hbm.at[pl.ds(c * N_NODES + off, WB_ROWS)], sb).wait()

    return k(flo, fhi, pk)


def _tc_body(x_ref, wt_ref, b_ref, o_ref):
    acc = jnp.dot(x_ref[0], wt_ref[:D_HALF, :],
                  preferred_element_type=jnp.float32)
    acc += jnp.dot(x_ref[1], wt_ref[D_HALF:, :],
                   preferred_element_type=jnp.float32)
    o_ref[...] = jnp.maximum(acc + b_ref[...], 0.0)


def _tc_linear_relu(agg2, wt, b2):
    blk = 2000
    grid = N_NODES // blk
    return pl.pallas_call(
        _tc_body,
        grid=(grid,),
        in_specs=[
            pl.BlockSpec((2, blk, D_HALF), lambda i: (0, i, 0)),
            pl.BlockSpec((2 * D_HALF, 2 * D_HALF), lambda i: (0, 0)),
            pl.BlockSpec((1, 2 * D_HALF), lambda i: (0, 0)),
        ],
        out_specs=pl.BlockSpec((blk, 2 * D_HALF), lambda i: (i, 0)),
        out_shape=jax.ShapeDtypeStruct((N_NODES, 2 * D_HALF), jnp.float32),
    )(agg2, wt, b2)


def kernel(feature, edge_index, W, b):
    src = edge_index[0].astype(jnp.int32)
    dst = edge_index[1].astype(jnp.int32)
    pk = (src | (dst << 16)).reshape(NS, N_EDGES // NS)
    pk = jnp.concatenate(
        [pk, jnp.full((NS, E_PAD), N_NODES << 16, jnp.int32)], axis=1)
    pk = pk.reshape(NS, NCHUNKS, CHUNK)
    flo = feature[:, :D_HALF]
    fhi = feature[:, D_HALF:]
    agg2 = _sc_gather_scatter(flo, fhi, pk)
    return _tc_linear_relu(agg2.reshape(NC, N_NODES, D_HALF), W.T,
                           b.reshape(1, 2 * D_HALF))
